# baseline (speedup 1.0000x reference)
import jax
import jax.numpy as jnp
from jax import lax
from jax.experimental import pallas as pl
from jax.experimental.pallas import tpu as pltpu
from jax.experimental.pallas import tpu_sc as plsc

NUM_CORES = 2
NUM_SUBCORES = 16
NUM_WORKERS = NUM_CORES * NUM_SUBCORES
NB = 4
LANES = 16


def kernel(x, table):
    batch, hist = x.shape
    vocab, d = table.shape
    idx = jnp.pad(x.astype(jnp.int32), ((0, 0), (0, 128 - hist)))
    table_p = jnp.concatenate([table, jnp.zeros_like(table)], axis=1)

    rows_per_worker = batch // NUM_WORKERS
    n_chunks = rows_per_worker // NB
    chunk = NB * hist
    assert batch % NUM_WORKERS == 0 and rows_per_worker % NB == 0
    assert n_chunks % 2 == 0

    mesh = plsc.VectorSubcoreMesh(core_axis_name="c", subcore_axis_name="s")

    @pl.kernel(
        out_type=jax.ShapeDtypeStruct((batch, hist, d), table.dtype),
        mesh=mesh,
        scratch_types=[
            pltpu.VMEM((NB, 128), jnp.int32),
            pltpu.VMEM((NB, 128), jnp.int32),
            pltpu.VMEM((chunk, 2 * d), jnp.float32),
            pltpu.VMEM((chunk, 2 * d), jnp.float32),
            pltpu.VMEM((chunk, d), jnp.float32),
            pltpu.VMEM((chunk, d), jnp.float32),
            pltpu.SemaphoreType.DMA,
            pltpu.SemaphoreType.DMA,
            pltpu.SemaphoreType.DMA,
            pltpu.SemaphoreType.DMA,
        ],
    )
    def gather_kernel(table_hbm, idx_hbm, out_hbm,
                      idx0, idx1, rows0, rows1, cmp0, cmp1,
                      sg0, sg1, sw0, sw1):
        wid = lax.axis_index("s") * NUM_CORES + lax.axis_index("c")
        row_base = wid * rows_per_worker
        idx_v = (idx0, idx1)
        rows_v = (rows0, rows1)
        cmp_v = (cmp0, cmp1)
        sg = (sg0, sg1)
        sw = (sw0, sw1)

        def fire_gathers(c, b):
            b0 = row_base + c * NB
            pltpu.sync_copy(idx_hbm.at[pl.ds(b0, NB)], idx_v[b])
            for j in range(NB):
                pltpu.async_copy(
                    table_hbm.at[idx_v[b].at[j, pl.ds(0, hist)]],
                    rows_v[b].at[pl.ds(j * hist, hist)],
                    sg[b],
                )

        def wait_gathers(b):
            for j in range(NB):
                pltpu.make_async_copy(
                    table_hbm.at[idx_v[b].at[j, pl.ds(0, hist)]],
                    rows_v[b].at[pl.ds(j * hist, hist)],
                    sg[b],
                ).wait()

        def wait_writes(b):
            for j in range(NB):
                pltpu.make_async_copy(
                    cmp_v[b].at[pl.ds(j * hist, hist)],
                    out_hbm.at[row_base + j],
                    sw[b],
                ).wait()

        def step(c, b):
            wait_gathers(b)

            @pl.when(c + 1 < n_chunks)
            def _():
                fire_gathers(c + 1, 1 - b)

            @pl.when(c >= 2)
            def _():
                wait_writes(b)

            @pl.loop(0, chunk)
            def _(r):
                for k in range(d // LANES):
                    sl = pl.ds(k * LANES, LANES)
                    cmp_v[b][pl.ds(r, 1), sl] = rows_v[b][pl.ds(r, 1), sl]

            b0 = row_base + c * NB
            for j in range(NB):
                pltpu.async_copy(
                    cmp_v[b].at[pl.ds(j * hist, hist)],
                    out_hbm.at[b0 + j],
                    sw[b],
                )

        fire_gathers(0, 0)

        @pl.loop(0, n_chunks, step=2)
        def _(c):
            step(c, 0)
            step(c + 1, 1)

        wait_writes(0)
        wait_writes(1)

    out = gather_kernel(table_p, idx)
    return out


# speedup vs baseline: 1.0010x; 1.0010x over previous
import jax
import jax.numpy as jnp
from jax import lax
from jax.experimental import pallas as pl
from jax.experimental.pallas import tpu as pltpu
from jax.experimental.pallas import tpu_sc as plsc

NUM_CORES = 2
NUM_SUBCORES = 16
NUM_WORKERS = NUM_CORES * NUM_SUBCORES
NB = 4
LANES = 16


def kernel(x, table):
    batch, hist = x.shape
    vocab, d = table.shape
    idx = jnp.pad(x.astype(jnp.int32), ((0, 0), (0, 128 - hist)))
    table_p = jnp.pad(table, ((0, 0), (0, d)))

    rows_per_worker = batch // NUM_WORKERS
    n_chunks = rows_per_worker // NB
    chunk = NB * hist
    assert batch % NUM_WORKERS == 0 and rows_per_worker % NB == 0
    assert n_chunks % 2 == 0

    mesh = plsc.VectorSubcoreMesh(core_axis_name="c", subcore_axis_name="s")

    @pl.kernel(
        out_type=jax.ShapeDtypeStruct((batch, hist, d), table.dtype),
        mesh=mesh,
        scratch_types=[
            pltpu.VMEM((NB, 128), jnp.int32),
            pltpu.VMEM((NB, 128), jnp.int32),
            pltpu.VMEM((chunk, 2 * d), jnp.float32),
            pltpu.VMEM((chunk, 2 * d), jnp.float32),
            pltpu.VMEM((chunk, d), jnp.float32),
            pltpu.VMEM((chunk, d), jnp.float32),
            pltpu.SemaphoreType.DMA,
            pltpu.SemaphoreType.DMA,
            pltpu.SemaphoreType.DMA,
            pltpu.SemaphoreType.DMA,
        ],
    )
    def gather_kernel(table_hbm, idx_hbm, out_hbm,
                      idx0, idx1, rows0, rows1, cmp0, cmp1,
                      sg0, sg1, sw0, sw1):
        wid = lax.axis_index("s") * NUM_CORES + lax.axis_index("c")
        row_base = wid * rows_per_worker
        idx_v = (idx0, idx1)
        rows_v = (rows0, rows1)
        cmp_v = (cmp0, cmp1)
        sg = (sg0, sg1)
        sw = (sw0, sw1)

        def fire_gathers(c, b):
            b0 = row_base + c * NB
            pltpu.sync_copy(idx_hbm.at[pl.ds(b0, NB)], idx_v[b])
            for j in range(NB):
                pltpu.async_copy(
                    table_hbm.at[idx_v[b].at[j, pl.ds(0, hist)]],
                    rows_v[b].at[pl.ds(j * hist, hist)],
                    sg[b],
                )

        def wait_gathers(b):
            for j in range(NB):
                pltpu.make_async_copy(
                    table_hbm.at[idx_v[b].at[j, pl.ds(0, hist)]],
                    rows_v[b].at[pl.ds(j * hist, hist)],
                    sg[b],
                ).wait()

        def wait_writes(b):
            for j in range(NB):
                pltpu.make_async_copy(
                    cmp_v[b].at[pl.ds(j * hist, hist)],
                    out_hbm.at[row_base + j],
                    sw[b],
                ).wait()

        def step(c, b):
            wait_gathers(b)

            @pl.when(c + 1 < n_chunks)
            def _():
                fire_gathers(c + 1, 1 - b)

            @pl.when(c >= 2)
            def _():
                wait_writes(b)

            @pl.loop(0, chunk)
            def _(r):
                for k in range(d // LANES):
                    sl = pl.ds(k * LANES, LANES)
                    cmp_v[b][pl.ds(r, 1), sl] = rows_v[b][pl.ds(r, 1), sl]

            b0 = row_base + c * NB
            for j in range(NB):
                pltpu.async_copy(
                    cmp_v[b].at[pl.ds(j * hist, hist)],
                    out_hbm.at[b0 + j],
                    sw[b],
                )

        fire_gathers(0, 0)

        @pl.loop(0, n_chunks, step=2)
        def _(c):
            step(c, 0)
            step(c + 1, 1)

        wait_writes(0)
        wait_writes(1)

    out = gather_kernel(table_p, idx)
    return out


# speedup vs baseline: 1.0287x; 1.0276x over previous
import jax
import jax.numpy as jnp
from jax import lax
from jax.experimental import pallas as pl
from jax.experimental.pallas import tpu as pltpu
from jax.experimental.pallas import tpu_sc as plsc

NUM_CORES = 2
NUM_SUBCORES = 16
NUM_WORKERS = NUM_CORES * NUM_SUBCORES
NB = 4
LANES = 16


def kernel(x, table):
    batch, hist = x.shape
    vocab, d = table.shape
    idx = jnp.pad(x.astype(jnp.int32), ((0, 0), (0, 128 - hist)))
    table_p = jnp.pad(table, ((0, 0), (0, d)))

    rows_per_worker = batch // NUM_WORKERS
    n_chunks = rows_per_worker // NB
    chunk = NB * hist
    assert batch % NUM_WORKERS == 0 and rows_per_worker % NB == 0
    assert n_chunks % 2 == 0

    mesh = plsc.VectorSubcoreMesh(core_axis_name="c", subcore_axis_name="s")

    @pl.kernel(
        out_type=jax.ShapeDtypeStruct((batch, hist * d), table.dtype),
        mesh=mesh,
        scratch_types=[
            pltpu.VMEM((NB, 128), jnp.int32),
            pltpu.VMEM((NB, 128), jnp.int32),
            pltpu.VMEM((chunk, 2 * d), jnp.float32),
            pltpu.VMEM((chunk, 2 * d), jnp.float32),
            pltpu.VMEM((NB, hist * d), jnp.float32),
            pltpu.VMEM((NB, hist * d), jnp.float32),
            pltpu.SemaphoreType.DMA,
            pltpu.SemaphoreType.DMA,
            pltpu.SemaphoreType.DMA,
            pltpu.SemaphoreType.DMA,
        ],
    )
    def gather_kernel(table_hbm, idx_hbm, out_hbm,
                      idx0, idx1, rows0, rows1, cmp0, cmp1,
                      sg0, sg1, sw0, sw1):
        wid = lax.axis_index("s") * NUM_CORES + lax.axis_index("c")
        row_base = wid * rows_per_worker
        idx_v = (idx0, idx1)
        rows_v = (rows0, rows1)
        cmp_v = (cmp0, cmp1)
        sg = (sg0, sg1)
        sw = (sw0, sw1)

        def fire_gathers(c, b):
            b0 = row_base + c * NB
            pltpu.sync_copy(idx_hbm.at[pl.ds(b0, NB)], idx_v[b])
            for j in range(NB):
                pltpu.async_copy(
                    table_hbm.at[idx_v[b].at[j, pl.ds(0, hist)]],
                    rows_v[b].at[pl.ds(j * hist, hist)],
                    sg[b],
                )

        def wait_gathers(b):
            for j in range(NB):
                pltpu.make_async_copy(
                    table_hbm.at[idx_v[b].at[j, pl.ds(0, hist)]],
                    rows_v[b].at[pl.ds(j * hist, hist)],
                    sg[b],
                ).wait()

        def wait_writes(b):
            pltpu.make_async_copy(
                cmp_v[b],
                out_hbm.at[pl.ds(row_base, NB)],
                sw[b],
            ).wait()

        def step(c, b):
            wait_gathers(b)

            @pl.when(c + 1 < n_chunks)
            def _():
                fire_gathers(c + 1, 1 - b)

            @pl.when(c >= 2)
            def _():
                wait_writes(b)

            for j in range(NB):
                @pl.loop(0, hist)
                def _(h):
                    for k in range(d // LANES):
                        cmp_v[b][pl.ds(j, 1), pl.ds(h * d + k * LANES, LANES)] = (
                            rows_v[b][pl.ds(j * hist + h, 1), pl.ds(k * LANES, LANES)]
                        )

            b0 = row_base + c * NB
            pltpu.async_copy(
                cmp_v[b],
                out_hbm.at[pl.ds(b0, NB)],
                sw[b],
            )

        fire_gathers(0, 0)

        @pl.loop(0, n_chunks, step=2)
        def _(c):
            step(c, 0)
            step(c + 1, 1)

        wait_writes(0)
        wait_writes(1)

    out = gather_kernel(table_p, idx)
    return out.reshape(batch, hist, d)
